# baseline (speedup 1.0000x reference)
import jax
import jax.numpy as jnp
from jax import lax
from jax.experimental import pallas as pl
from jax.experimental.pallas import tpu as pltpu
from jax.experimental.pallas import tpu_sc as plsc

N = 10000
E = 160000
D = 256
HID = 512
HALF = D // 2
L = 16
NC = 2
NS = 16
EPT = E // NS
B = 80
C = EPT // B
R = 3
ROWS_PT = N // NS
IB = 80
IB_TAIL = ROWS_PT - (ROWS_PT // IB) * IB


def _spmm_body(table, epack, vals, bias, out,
               idxb, eb0, eb1, eb2, valsb, rows0, rows1, rows2, bias_v,
               acc, *sems):
    rows = (rows0, rows1, rows2)
    ebuf = (eb0, eb1, eb2)
    c = lax.axis_index("c")
    s = lax.axis_index("s")
    base = s * ROWS_PT
    sg = sems[0:3]
    ss = sems[3:6]
    se = sems[6:9]
    sv = sems[9:12]

    def prefetch(k, j):
        pltpu.async_copy(epack.at[s, k], ebuf[j], se[j])
        pltpu.async_copy(vals.at[s, k], valsb.at[j], sv[j])

    def gather_issue(k, j):
        pltpu.make_async_copy(epack.at[s, 0], ebuf[j], se[j]).wait()
        for jj in range(B // L):
            sl = pl.ds(jj * L, L)
            idxb[j, sl] = ebuf[j][1, sl] + c
        pltpu.async_copy(table.at[idxb.at[j]], rows[j], sg[j])

    def scatter_wait(j):
        pltpu.make_async_copy(rows[j], acc.at[ebuf[j].at[0]], ss[j]).wait()

    def process(k, j):
        pltpu.make_async_copy(table.at[idxb.at[j]], rows[j], sg[j]).wait()

        pltpu.make_async_copy(vals.at[s, 0], valsb.at[j], sv[j]).wait()

        @pl.loop(0, B // L)
        def _scale(gi):
            vv = valsb[j, pl.ds(gi * L, L)]
            for i in range(L):
                r = gi * L + i
                v = vv[i]
                for jj in range(HALF // L):
                    sl = pl.ds(jj * L, L)
                    rows[j][r, sl] = rows[j][r, sl] * v

        pltpu.async_copy(rows[j], acc.at[ebuf[j].at[0]], ss[j], add=True)

    def emit_step(k, m, swait, pref, gissue):
        if swait:
            scatter_wait((m + 2) % R)
        if pref:
            prefetch(k + 2, (m + 2) % R)
        if gissue:
            gather_issue(k + 1, (m + 1) % R)
        process(k, m)

    prefetch(0, 0)
    prefetch(1, 1)
    gather_issue(0, 0)

    pltpu.sync_copy(bias.at[c], bias_v)

    @pl.loop(0, IB)
    def _fill(r):
        for j in range(HALF // L):
            sl = pl.ds(j * L, L)
            rows1[r, sl] = bias_v[sl]

    for kk in range(ROWS_PT // IB):
        pltpu.sync_copy(rows1, acc.at[pl.ds(base + kk * IB, IB)])
    if IB_TAIL:
        pltpu.sync_copy(
            rows1.at[pl.ds(0, IB_TAIL)],
            acc.at[pl.ds(base + (ROWS_PT // IB) * IB, IB_TAIL)])

    plsc.subcore_barrier()

    for k in range(R):
        emit_step(k, k, k >= 1, True, True)

    NG = (C - R - 2) // R

    @pl.loop(1, NG + 1)
    def _steady(g):
        for jj in range(R):
            emit_step(R * g + jj, jj, True, True, True)

    for k in range(R * (NG + 1), C):
        emit_step(k, k % R, True, k + 2 < C, k + 1 < C)
    scatter_wait((C - 1) % R)

    plsc.subcore_barrier()

    for kk in range(ROWS_PT // IB):
        sl = pl.ds(base + kk * IB, IB)
        pltpu.sync_copy(acc.at[sl], out.at[sl, c])
    if IB_TAIL:
        sl = pl.ds(base + (ROWS_PT // IB) * IB, IB_TAIL)
        pltpu.sync_copy(acc.at[sl], out.at[sl, c])


def _spmm(table, epack, vals, bias):
    mesh = plsc.VectorSubcoreMesh(core_axis_name="c", subcore_axis_name="s")
    return pl.kernel(
        _spmm_body,
        out_type=jax.ShapeDtypeStruct((N, NC, HALF), jnp.float32),
        mesh=mesh,
        scratch_types=[
            pltpu.VMEM((R, B), jnp.int32),
            pltpu.VMEM((2, B), jnp.int32),
            pltpu.VMEM((2, B), jnp.int32),
            pltpu.VMEM((2, B), jnp.int32),
            pltpu.VMEM((R, B), jnp.float32),
            pltpu.VMEM((B, HALF), jnp.float32),
            pltpu.VMEM((B, HALF), jnp.float32),
            pltpu.VMEM((B, HALF), jnp.float32),
            pltpu.VMEM((HALF,), jnp.float32),
            pltpu.VMEM_SHARED((N, HALF), jnp.float32),
        ] + [pltpu.SemaphoreType.DMA] * 12,
    )(table, epack, vals, bias)


def _dense_body(s_ref, w1_ref, b1_ref, w2_ref, o_ref):
    a = s_ref[...].reshape(-1, D)
    h = jnp.dot(a, w1_ref[...], preferred_element_type=jnp.float32)
    h = jnp.maximum(h + b1_ref[...], 0.0)
    g = jnp.dot(h, w2_ref[...], preferred_element_type=jnp.float32)
    o_ref[...] = g.reshape(-1, NC, HALF)


def _dense(s3, W1, b1, W2):
    M = 1000
    return pl.pallas_call(
        _dense_body,
        grid=(N // M,),
        in_specs=[
            pl.BlockSpec((M, NC, HALF), lambda i: (i, 0, 0)),
            pl.BlockSpec((D, HID), lambda i: (0, 0)),
            pl.BlockSpec((1, HID), lambda i: (0, 0)),
            pl.BlockSpec((HID, D), lambda i: (0, 0)),
        ],
        out_specs=pl.BlockSpec((M, NC, HALF), lambda i: (i, 0, 0)),
        out_shape=jax.ShapeDtypeStruct((N, NC, HALF), jnp.float32),
    )(s3, W1, b1.reshape(1, HID), W2)


def kernel(x, adj_vals, edge_index, W1, b1, W2, b2):
    src = edge_index[0].astype(jnp.int32)
    dst = edge_index[1].astype(jnp.int32)
    epack = jnp.stack([dst.reshape(NS, C, B),
                       (src * 2).reshape(NS, C, B)], axis=2)
    vals = adj_vals.reshape(NS, C, B)

    zero_bias = jnp.zeros((NC, HALF), jnp.float32)
    s3 = _spmm(x.reshape(N * NC, HALF), epack, vals, zero_bias)
    g3 = _dense(s3, W1, b1, W2)
    out3 = _spmm(g3.reshape(N * NC, HALF), epack, vals,
                 b2.reshape(NC, HALF))
    return out3.reshape(N, D)


# speedup vs baseline: 1.0137x; 1.0137x over previous
import jax
import jax.numpy as jnp
from jax import lax
from jax.experimental import pallas as pl
from jax.experimental.pallas import tpu as pltpu
from jax.experimental.pallas import tpu_sc as plsc

N = 10000
E = 160000
D = 256
HID = 512
HALF = D // 2
L = 16
NC = 2
NS = 16
EPT = E // NS
B = 80
C = EPT // B
R = 3
ROWS_PT = N // NS
IB = 80
IB_TAIL = ROWS_PT - (ROWS_PT // IB) * IB


def _spmm_body(table, srcs, dsts, vals, bias, out,
               idxb, dstb, valsb, rows0, rows1, rows2, bias_v, acc,
               *sems):
    rows = (rows0, rows1, rows2)
    c = lax.axis_index("c")
    s = lax.axis_index("s")
    base = s * ROWS_PT
    sg = sems[0:3]
    ss = sems[3:6]
    si = sems[6:9]
    sd = sems[9:12]
    sv = sems[12:15]

    def prefetch(k, j):
        pltpu.async_copy(srcs.at[s, k], idxb.at[j], si[j])
        pltpu.async_copy(dsts.at[s, k], dstb.at[j], sd[j])
        pltpu.async_copy(vals.at[s, k], valsb.at[j], sv[j])

    def gather_issue(k, j):
        pltpu.make_async_copy(srcs.at[s, 0], idxb.at[j], si[j]).wait()
        for jj in range(B // L):
            sl = pl.ds(jj * L, L)
            idxb[j, sl] = idxb[j, sl] * 2 + c
        pltpu.async_copy(table.at[idxb.at[j]], rows[j], sg[j])

    def scatter_wait(j):
        pltpu.make_async_copy(rows[j], acc.at[dstb.at[j]], ss[j]).wait()

    def process(k, j):
        pltpu.make_async_copy(table.at[idxb.at[j]], rows[j], sg[j]).wait()
        pltpu.make_async_copy(vals.at[s, 0], valsb.at[j], sv[j]).wait()

        @pl.loop(0, B // L)
        def _scale(gi):
            vv = valsb[j, pl.ds(gi * L, L)]
            for i in range(L):
                r = gi * L + i
                v = vv[i]
                for jj in range(HALF // L):
                    sl = pl.ds(jj * L, L)
                    rows[j][r, sl] = rows[j][r, sl] * v

        pltpu.make_async_copy(dsts.at[s, 0], dstb.at[j], sd[j]).wait()
        pltpu.async_copy(rows[j], acc.at[dstb.at[j]], ss[j], add=True)

    def emit_step(k, m, swait, pref, gissue):
        if swait:
            scatter_wait((m + 2) % R)
        if pref:
            prefetch(k + 2, (m + 2) % R)
        if gissue:
            gather_issue(k + 1, (m + 1) % R)
        process(k, m)

    prefetch(0, 0)
    prefetch(1, 1)
    gather_issue(0, 0)

    pltpu.sync_copy(bias.at[c], bias_v)

    @pl.loop(0, IB)
    def _fill(r):
        for j in range(HALF // L):
            sl = pl.ds(j * L, L)
            rows1[r, sl] = bias_v[sl]

    for kk in range(ROWS_PT // IB):
        pltpu.sync_copy(rows1, acc.at[pl.ds(base + kk * IB, IB)])
    if IB_TAIL:
        pltpu.sync_copy(
            rows1.at[pl.ds(0, IB_TAIL)],
            acc.at[pl.ds(base + (ROWS_PT // IB) * IB, IB_TAIL)])

    plsc.subcore_barrier()

    for k in range(R):
        emit_step(k, k, k >= 1, True, True)

    NG = (C - R - 2) // R

    @pl.loop(1, NG + 1)
    def _steady(g):
        for jj in range(R):
            emit_step(R * g + jj, jj, True, True, True)

    for k in range(R * (NG + 1), C):
        emit_step(k, k % R, True, k + 2 < C, k + 1 < C)
    scatter_wait((C - 1) % R)

    plsc.subcore_barrier()

    for kk in range(ROWS_PT // IB):
        sl = pl.ds(base + kk * IB, IB)
        pltpu.sync_copy(acc.at[sl], out.at[sl, c])
    if IB_TAIL:
        sl = pl.ds(base + (ROWS_PT // IB) * IB, IB_TAIL)
        pltpu.sync_copy(acc.at[sl], out.at[sl, c])


def _spmm(table, srcs, dsts, vals, bias):
    mesh = plsc.VectorSubcoreMesh(core_axis_name="c", subcore_axis_name="s")
    return pl.kernel(
        _spmm_body,
        out_type=jax.ShapeDtypeStruct((N, NC, HALF), jnp.float32),
        mesh=mesh,
        scratch_types=[
            pltpu.VMEM((R, B), jnp.int32),
            pltpu.VMEM((R, B), jnp.int32),
            pltpu.VMEM((R, B), jnp.float32),
            pltpu.VMEM((B, HALF), jnp.float32),
            pltpu.VMEM((B, HALF), jnp.float32),
            pltpu.VMEM((B, HALF), jnp.float32),
            pltpu.VMEM((HALF,), jnp.float32),
            pltpu.VMEM_SHARED((N, HALF), jnp.float32),
        ] + [pltpu.SemaphoreType.DMA] * 15,
    )(table, srcs, dsts, vals, bias)


def _dense_body(s_ref, w1_ref, b1_ref, w2_ref, o_ref):
    a = s_ref[...].reshape(-1, D)
    h = jnp.dot(a, w1_ref[...], preferred_element_type=jnp.float32)
    h = jnp.maximum(h + b1_ref[...], 0.0)
    g = jnp.dot(h, w2_ref[...], preferred_element_type=jnp.float32)
    o_ref[...] = g.reshape(-1, NC, HALF)


def _dense(s3, W1, b1, W2):
    M = 1000
    return pl.pallas_call(
        _dense_body,
        grid=(N // M,),
        in_specs=[
            pl.BlockSpec((M, NC, HALF), lambda i: (i, 0, 0)),
            pl.BlockSpec((D, HID), lambda i: (0, 0)),
            pl.BlockSpec((1, HID), lambda i: (0, 0)),
            pl.BlockSpec((HID, D), lambda i: (0, 0)),
        ],
        out_specs=pl.BlockSpec((M, NC, HALF), lambda i: (i, 0, 0)),
        out_shape=jax.ShapeDtypeStruct((N, NC, HALF), jnp.float32),
    )(s3, W1, b1.reshape(1, HID), W2)


def kernel(x, adj_vals, edge_index, W1, b1, W2, b2):
    src = edge_index[0].astype(jnp.int32)
    dst = edge_index[1].astype(jnp.int32)
    srcs = src.reshape(NS, C, B)
    dsts = dst.reshape(NS, C, B)
    vals = adj_vals.reshape(NS, C, B)

    zero_bias = jnp.zeros((NC, HALF), jnp.float32)
    s3 = _spmm(x.reshape(N * NC, HALF), srcs, dsts, vals, zero_bias)
    g3 = _dense(s3, W1, b1, W2)
    out3 = _spmm(g3.reshape(N * NC, HALF), srcs, dsts, vals,
                 b2.reshape(NC, HALF))
    return out3.reshape(N, D)
